# Initial kernel scaffold; baseline (speedup 1.0000x reference)
import jax
import jax.numpy as jnp
from jax.experimental import pallas as pl


def kernel(inputs, token_table, pos_table):
    raise NotImplementedError("write your pallas kernel here")


# speedup vs baseline: 2.8837x; 2.8837x over previous
import functools

import jax
import jax.numpy as jnp
from jax import lax
from jax.experimental import pallas as pl
from jax.experimental.pallas import tpu as pltpu
from jax.experimental.pallas import tpu_sc as plsc

BATCH = 4096
SEQ = 200
EMB = 64
NC = 2
NS = 16
NW = NC * NS
ROWS = BATCH * SEQ
RPW = ROWS // NW
CH = 400
NCH = RPW // CH
SG = 80
NSG = CH // SG
LANES = 16


def _sc_embed(idx_flat, token_table, pos_table):
    mesh = plsc.VectorSubcoreMesh(core_axis_name="c", subcore_axis_name="s")

    @functools.partial(
        pl.kernel,
        mesh=mesh,
        compiler_params=pltpu.CompilerParams(use_tc_tiling_on_sc=False),
        out_type=jax.ShapeDtypeStruct((ROWS, EMB), jnp.float32),
        scratch_types=[
            pltpu.VMEM((CH,), jnp.int32),
            pltpu.VMEM((CH, EMB), jnp.float32),
            pltpu.VMEM((CH, EMB), jnp.float32),
            pltpu.SemaphoreType.DMA,
        ],
    )
    def k(idx_hbm, tab_hbm, pos_hbm, out_hbm, idx_v, tok_v, pos_v, sem):
        wid = lax.axis_index("s") * NC + lax.axis_index("c")
        base = wid * RPW
        for r in range(CH // SEQ):
            pltpu.sync_copy(pos_hbm, pos_v.at[pl.ds(r * SEQ, SEQ)])

        def chunk_body(g, carry):
            start = base + g * CH
            pltpu.sync_copy(idx_hbm.at[pl.ds(start, CH)], idx_v)
            for j in range(NSG):
                pltpu.async_copy(
                    tab_hbm.at[idx_v.at[pl.ds(j * SG, SG)]],
                    tok_v.at[pl.ds(j * SG, SG)],
                    sem,
                ).wait()

            def row_body(r, c2):
                for c in range(EMB // LANES):
                    sl = pl.ds(c * LANES, LANES)
                    tok_v[r, sl] = tok_v[r, sl] + pos_v[r, sl]
                return c2

            lax.fori_loop(0, CH, row_body, 0)
            pltpu.sync_copy(tok_v, out_hbm.at[pl.ds(start, CH)])
            return carry

        lax.fori_loop(0, NCH, chunk_body, 0)

    return k(idx_flat, token_table, pos_table)


def kernel(inputs, token_table, pos_table):
    idx_flat = inputs.reshape(ROWS).astype(jnp.int32)
    out = _sc_embed(idx_flat, token_table, pos_table)
    return out.reshape(BATCH, SEQ, EMB)
